# baseline (speedup 1.0000x reference)
import jax
import jax.numpy as jnp
from jax.experimental import pallas as pl
from jax.experimental.pallas import tpu as pltpu

_RADIUS = 2
_H = 224
_W = 224
_N = _H * _W
_K = 150
_C = 96
_B = 2

_OFFSETS = tuple(
    (dx, dy)
    for dx in range(-_RADIUS, _RADIUS + 1)
    for dy in range(-_RADIUS, _RADIUS + 1)
    if dx * dx + dy * dy < _RADIUS * _RADIUS
)
_POS = tuple((dx, dy) for (dx, dy) in _OFFSETS if dx > 0 or (dx == 0 and dy > 0))

_RB = 56
_NRB = _H // _RB
_CB = 32
_NCB = _C // _CB


def _croll(x, s):
    return x if s == 0 else jnp.roll(x, s, axis=1)


def _kernel(f_ref, s_ref, stop_ref, sbot_ref, out_ref, corr_ref, w_ref,
            acc_ref):
    b = pl.program_id(0)
    c = pl.program_id(1)

    @pl.when(c == 0)
    def _init():
        corr_ref[...] = jnp.zeros_like(corr_ref)
        w_ref[...] = jnp.zeros_like(w_ref)

    @pl.when((b == 0) & (c == 0))
    def _init_acc():
        acc_ref[...] = jnp.zeros_like(acc_ref)

    @pl.when(c < _NCB)
    def _stencil():
        f = f_ref[0]
        fcol = jnp.roll(f, -1, axis=2)
        frow = jnp.roll(f, -1, axis=1)
        frowcol = jnp.roll(fcol, -1, axis=1)
        corr_ref[0] += jnp.sum(f * f, axis=0)
        corr_ref[1] += jnp.sum(f * fcol, axis=0)
        corr_ref[2] += jnp.sum(f * frow, axis=0)
        corr_ref[3] += jnp.sum(f * frowcol, axis=0)
        corr_ref[4] += jnp.sum(fcol * frow, axis=0)

    @pl.when(c == _NCB - 1)
    def _finalize_w():
        sq = corr_ref[0]
        acc_of = {(0, 1): 1, (1, 0): 2, (1, 1): 3, (1, -1): 4}
        for fidx, (dx, dy) in enumerate(_POS):
            r0, r1 = max(0, -dx), _H - max(0, dx)
            c0, c1 = max(0, -dy), _W - max(0, dy)
            if (dx, dy) == (1, -1):
                d = corr_ref[4][r0:r1, c0 - 1:c1 - 1]
            else:
                d = corr_ref[acc_of[(dx, dy)]][r0:r1, c0:c1]
            fsq = (sq[r0:r1, c0:c1] + sq[r0 + dx:r1 + dx, c0 + dy:c1 + dy]
                   - 2.0 * d)
            w_ref[fidx, 1 + r0:1 + r1, c0:c1] = jnp.exp(
                -float(dx * dx + dy * dy) - fsq)

    @pl.when(c >= _NCB)
    def _reduce():
        i = jnp.maximum(c - _NCB, 0)
        ss_mid = jnp.concatenate(
            [jnp.sum(s_ref[0, g:g + 8], axis=-1) for g in range(0, _RB, 8)],
            axis=0)
        ss_top = jnp.sum(stop_ref[0], axis=-1)
        ss_bot = jnp.sum(sbot_ref[0], axis=-1)
        ss_ext = jnp.concatenate([ss_top, ss_mid, ss_bot], axis=0)

        v = ss_mid
        deg = jnp.ones((_RB, _W), jnp.float32)
        row0 = pl.multiple_of(i * _RB, 8)
        for fidx, (dx, dy) in enumerate(_POS):
            w_ext = w_ref[fidx, pl.ds(row0, _RB + 2), :]
            w_here = w_ext[1:_RB + 1]
            w_mir = _croll(w_ext[1 - dx:_RB + 1 - dx], dy)
            v = v + w_here * _croll(ss_ext[1 + dx:_RB + 1 + dx], -dy)
            v = v + w_mir * _croll(ss_ext[1 - dx:_RB + 1 - dx], dy)
            deg = deg + w_here + w_mir

        part = jnp.zeros((2, _K), jnp.float32)
        for r in range(_RB):
            lhs_r = jnp.concatenate([v[r:r + 1], deg[r:r + 1]], axis=0)
            part = part + jax.lax.dot_general(
                lhs_r, s_ref[0, r], (((1,), (0,)), ((), ())),
                preferred_element_type=jnp.float32)
        acc_ref[b] += part

    @pl.when((b == _B - 1) & (c == _NCB + _NRB - 1))
    def _finalize():
        acc = acc_ref[...]
        out_ref[0, 0] = jnp.sum(acc[:, 0, :] / acc[:, 1, :])


def kernel(classification, features):
    S4 = classification.reshape(_B, _H, _W, _K)

    def _i(c):
        return jnp.minimum(jnp.maximum(c - _NCB, 0), _NRB - 1)

    loss = pl.pallas_call(
        _kernel,
        grid=(_B, _NCB + _NRB),
        in_specs=[
            pl.BlockSpec((1, _CB, _H, _W),
                         lambda b, c: (b, jnp.minimum(c, _NCB - 1), 0, 0)),
            pl.BlockSpec((1, _RB, _W, _K), lambda b, c: (b, _i(c), 0, 0)),
            pl.BlockSpec((1, 1, _W, _K),
                         lambda b, c: (b, jnp.maximum(_i(c) * _RB - 1, 0), 0, 0)),
            pl.BlockSpec((1, 1, _W, _K),
                         lambda b, c: (b, jnp.minimum((_i(c) + 1) * _RB, _H - 1),
                                       0, 0)),
        ],
        out_specs=pl.BlockSpec(memory_space=pltpu.SMEM),
        out_shape=jax.ShapeDtypeStruct((1, 1), jnp.float32),
        scratch_shapes=[
            pltpu.VMEM((5, _H, _W), jnp.float32),
            pltpu.VMEM((len(_POS), _H + 2, _W), jnp.float32),
            pltpu.VMEM((_B, 2, _K), jnp.float32),
        ],
    )(features, S4, S4, S4)

    return loss[0, 0]


# speedup vs baseline: 1.1855x; 1.1855x over previous
import jax
import jax.numpy as jnp
from jax.experimental import pallas as pl
from jax.experimental.pallas import tpu as pltpu

_RADIUS = 2
_H = 224
_W = 224
_N = _H * _W
_K = 150
_C = 96
_B = 2

_OFFSETS = tuple(
    (dx, dy)
    for dx in range(-_RADIUS, _RADIUS + 1)
    for dy in range(-_RADIUS, _RADIUS + 1)
    if dx * dx + dy * dy < _RADIUS * _RADIUS
)
_POS = tuple((dx, dy) for (dx, dy) in _OFFSETS if dx > 0 or (dx == 0 and dy > 0))

_RB = 56
_NRB = _H // _RB
_CB = 32
_NCB = _C // _CB
_BN = _RB * _W


def _stencil_kernel(f_ref, out_ref, acc_ref):
    c = pl.program_id(1)

    @pl.when(c == 0)
    def _init():
        acc_ref[...] = jnp.zeros_like(acc_ref)

    f = f_ref[0]
    fcol = jnp.roll(f, -1, axis=2)
    frow = jnp.roll(f, -1, axis=1)
    frowcol = jnp.roll(fcol, -1, axis=1)
    acc_ref[0] += jnp.sum(f * f, axis=0)
    acc_ref[1] += jnp.sum(f * fcol, axis=0)
    acc_ref[2] += jnp.sum(f * frow, axis=0)
    acc_ref[3] += jnp.sum(f * frowcol, axis=0)
    acc_ref[4] += jnp.sum(fcol * frow, axis=0)

    @pl.when(c == _NCB - 1)
    def _finalize():
        sq = acc_ref[0]
        acc_of = {(0, 1): 1, (1, 0): 2, (1, 1): 3, (1, -1): 4}
        for i, (dx, dy) in enumerate(_POS):
            r0, r1 = max(0, -dx), _H - max(0, dx)
            c0, c1 = max(0, -dy), _W - max(0, dy)
            if (dx, dy) == (1, -1):
                d = acc_ref[4][r0:r1, c0 - 1:c1 - 1]
            else:
                d = acc_ref[acc_of[(dx, dy)]][r0:r1, c0:c1]
            fsq = (sq[r0:r1, c0:c1] + sq[r0 + dx:r1 + dx, c0 + dy:c1 + dy]
                   - 2.0 * d)
            w = jnp.exp(-float(dx * dx + dy * dy) - fsq)
            out_ref[0, i] = jnp.zeros((_H, _W), jnp.float32)
            out_ref[0, i, r0:r1, c0:c1] = w


def _croll(x, s):
    return x if s == 0 else jnp.roll(x, s, axis=1)


def _fused_kernel(s_ref, stop_ref, sbot_ref, wprev_ref, wcur_ref, wnext_ref,
                  out_ref, acc_ref):
    b = pl.program_id(0)
    i = pl.program_id(1)

    @pl.when((b == 0) & (i == 0))
    def _init():
        acc_ref[...] = jnp.zeros_like(acc_ref)

    ss_mid = jnp.concatenate(
        [jnp.sum(s_ref[0, g:g + 8], axis=-1) for g in range(0, _RB, 8)],
        axis=0)
    ss_top = jnp.sum(stop_ref[0], axis=-1)
    ss_bot = jnp.sum(sbot_ref[0], axis=-1)
    ss_ext = jnp.concatenate([ss_top, ss_mid, ss_bot], axis=0)

    gate_top = jnp.where(i > 0, 1.0, 0.0)
    gate_bot = jnp.where(i < _NRB - 1, 1.0, 0.0)

    v = ss_mid
    deg = jnp.ones((_RB, _W), jnp.float32)
    for f, (dx, dy) in enumerate(_POS):
        w_ext = jnp.concatenate(
            [wprev_ref[0, f, 0, _RB - 1:_RB, :] * gate_top,
             wcur_ref[0, f, 0],
             wnext_ref[0, f, 0, 0:1, :] * gate_bot],
            axis=0)
        w_here = w_ext[1:_RB + 1]
        w_mir = _croll(w_ext[1 - dx:_RB + 1 - dx], dy)
        v = v + w_here * _croll(ss_ext[1 + dx:_RB + 1 + dx], -dy)
        v = v + w_mir * _croll(ss_ext[1 - dx:_RB + 1 - dx], dy)
        deg = deg + w_here + w_mir

    part = jnp.zeros((2, _K), jnp.float32)
    for r in range(_RB):
        lhs_r = jnp.concatenate([v[r:r + 1], deg[r:r + 1]], axis=0)
        part = part + jax.lax.dot_general(
            lhs_r, s_ref[0, r], (((1,), (0,)), ((), ())),
            preferred_element_type=jnp.float32)
    acc_ref[b] += part

    @pl.when((b == _B - 1) & (i == _NRB - 1))
    def _finalize():
        acc = acc_ref[...]
        out_ref[0, 0] = jnp.sum(acc[:, 0, :] / acc[:, 1, :])


def kernel(classification, features):
    S4 = classification.reshape(_B, _H, _W, _K)

    wf = pl.pallas_call(
        _stencil_kernel,
        grid=(_B, _NCB),
        in_specs=[pl.BlockSpec((1, _CB, _H, _W), lambda b, c: (b, c, 0, 0))],
        out_specs=pl.BlockSpec((1, len(_POS), _H, _W), lambda b, c: (b, 0, 0, 0)),
        out_shape=jax.ShapeDtypeStruct((_B, len(_POS), _H, _W), jnp.float32),
        scratch_shapes=[pltpu.VMEM((5, _H, _W), jnp.float32)],
    )(features)

    wfr = wf.reshape(_B, len(_POS), _NRB, _RB, _W)

    loss = pl.pallas_call(
        _fused_kernel,
        grid=(_B, _NRB),
        in_specs=[
            pl.BlockSpec((1, _RB, _W, _K), lambda b, i: (b, i, 0, 0)),
            pl.BlockSpec((1, 1, _W, _K),
                         lambda b, i: (b, jnp.maximum(i * _RB - 1, 0), 0, 0)),
            pl.BlockSpec((1, 1, _W, _K),
                         lambda b, i: (b, jnp.minimum((i + 1) * _RB, _H - 1), 0, 0)),
            pl.BlockSpec((1, len(_POS), 1, _RB, _W),
                         lambda b, i: (b, 0, jnp.maximum(i - 1, 0), 0, 0)),
            pl.BlockSpec((1, len(_POS), 1, _RB, _W), lambda b, i: (b, 0, i, 0, 0)),
            pl.BlockSpec((1, len(_POS), 1, _RB, _W),
                         lambda b, i: (b, 0, jnp.minimum(i + 1, _NRB - 1), 0, 0)),
        ],
        out_specs=pl.BlockSpec(memory_space=pltpu.SMEM),
        out_shape=jax.ShapeDtypeStruct((1, 1), jnp.float32),
        scratch_shapes=[pltpu.VMEM((_B, 2, _K), jnp.float32)],
    )(S4, S4, S4, wfr, wfr, wfr)

    return loss[0, 0]


# speedup vs baseline: 1.1950x; 1.0080x over previous
import jax
import jax.numpy as jnp
from jax.experimental import pallas as pl
from jax.experimental.pallas import tpu as pltpu

_RADIUS = 2
_H = 224
_W = 224
_N = _H * _W
_K = 150
_C = 96
_B = 2

_OFFSETS = tuple(
    (dx, dy)
    for dx in range(-_RADIUS, _RADIUS + 1)
    for dy in range(-_RADIUS, _RADIUS + 1)
    if dx * dx + dy * dy < _RADIUS * _RADIUS
)
_POS = tuple((dx, dy) for (dx, dy) in _OFFSETS if dx > 0 or (dx == 0 and dy > 0))

_RB = 56
_NRB = _H // _RB
_CB = 32
_NCB = _C // _CB
_BN = _RB * _W


def _stencil_kernel(f_ref, out_ref, acc_ref):
    c = pl.program_id(1)

    @pl.when(c == 0)
    def _init():
        acc_ref[...] = jnp.zeros_like(acc_ref)

    f = f_ref[0]
    fcol = jnp.roll(f, -1, axis=2)
    frow = jnp.roll(f, -1, axis=1)
    frowcol = jnp.roll(fcol, -1, axis=1)
    acc_ref[0] += jnp.sum(f * f, axis=0)
    acc_ref[1] += jnp.sum(f * fcol, axis=0)
    acc_ref[2] += jnp.sum(f * frow, axis=0)
    acc_ref[3] += jnp.sum(f * frowcol, axis=0)
    acc_ref[4] += jnp.sum(fcol * frow, axis=0)

    @pl.when(c == _NCB - 1)
    def _finalize():
        sq = acc_ref[0]
        acc_of = {(0, 1): 1, (1, 0): 2, (1, 1): 3, (1, -1): 4}
        for i, (dx, dy) in enumerate(_POS):
            r0, r1 = max(0, -dx), _H - max(0, dx)
            c0, c1 = max(0, -dy), _W - max(0, dy)
            if (dx, dy) == (1, -1):
                d = acc_ref[4][r0:r1, c0 - 1:c1 - 1]
            else:
                d = acc_ref[acc_of[(dx, dy)]][r0:r1, c0:c1]
            fsq = (sq[r0:r1, c0:c1] + sq[r0 + dx:r1 + dx, c0 + dy:c1 + dy]
                   - 2.0 * d)
            w = jnp.exp(-float(dx * dx + dy * dy) - fsq)
            out_ref[0, i] = jnp.zeros((_H, _W), jnp.float32)
            out_ref[0, i, r0:r1, c0:c1] = w


def _croll(x, s):
    return x if s == 0 else jnp.roll(x, s, axis=1)


def _fused_kernel(s_ref, sbot_ref, wx_ref, out_ref, carry_ref, acc_ref):
    b = pl.program_id(0)
    i = pl.program_id(1)

    @pl.when((b == 0) & (i == 0))
    def _init():
        acc_ref[...] = jnp.zeros_like(acc_ref)
        carry_ref[...] = jnp.zeros_like(carry_ref)

    ss_mid = jnp.concatenate(
        [jnp.sum(s_ref[0, g:g + 8], axis=-1) for g in range(0, _RB, 8)],
        axis=0)
    ss_top = carry_ref[...]
    ss_bot = jnp.sum(sbot_ref[0], axis=-1)
    ss_ext = jnp.concatenate([ss_top, ss_mid, ss_bot], axis=0)
    carry_ref[...] = ss_mid[_RB - 1:_RB]

    v = ss_mid
    deg = jnp.ones((_RB, _W), jnp.float32)
    for f, (dx, dy) in enumerate(_POS):
        w_ext = wx_ref[0, f, 0, 0:_RB + 2]
        w_here = w_ext[1:_RB + 1]
        w_mir = _croll(w_ext[1 - dx:_RB + 1 - dx], dy)
        v = v + w_here * _croll(ss_ext[1 + dx:_RB + 1 + dx], -dy)
        v = v + w_mir * _croll(ss_ext[1 - dx:_RB + 1 - dx], dy)
        deg = deg + w_here + w_mir

    part = jnp.zeros((2, _K), jnp.float32)
    for r in range(_RB):
        lhs_r = jnp.concatenate([v[r:r + 1], deg[r:r + 1]], axis=0)
        part = part + jax.lax.dot_general(
            lhs_r, s_ref[0, r], (((1,), (0,)), ((), ())),
            preferred_element_type=jnp.float32)
    acc_ref[b] += part

    @pl.when((b == _B - 1) & (i == _NRB - 1))
    def _finalize():
        acc = acc_ref[...]
        out_ref[0, 0] = jnp.sum(acc[:, 0, :] / acc[:, 1, :])


def kernel(classification, features):
    S4 = classification.reshape(_B, _H, _W, _K)

    wf = pl.pallas_call(
        _stencil_kernel,
        grid=(_B, _NCB),
        in_specs=[pl.BlockSpec((1, _CB, _H, _W), lambda b, c: (b, c, 0, 0))],
        out_specs=pl.BlockSpec((1, len(_POS), _H, _W), lambda b, c: (b, 0, 0, 0)),
        out_shape=jax.ShapeDtypeStruct((_B, len(_POS), _H, _W), jnp.float32),
        scratch_shapes=[pltpu.VMEM((5, _H, _W), jnp.float32)],
    )(features)

    wfp = jnp.pad(wf, ((0, 0), (0, 0), (1, 1), (0, 0)))
    wx = jnp.stack(
        [wfp[:, :, i * _RB:i * _RB + _RB + 2, :] for i in range(_NRB)],
        axis=2)
    wx = jnp.pad(wx, ((0, 0), (0, 0), (0, 0), (0, 64 - (_RB + 2)), (0, 0)))

    loss = pl.pallas_call(
        _fused_kernel,
        grid=(_B, _NRB),
        in_specs=[
            pl.BlockSpec((1, _RB, _W, _K), lambda b, i: (b, i, 0, 0)),
            pl.BlockSpec((1, 1, _W, _K),
                         lambda b, i: (b, jnp.minimum((i + 1) * _RB, _H - 1), 0, 0)),
            pl.BlockSpec((1, len(_POS), 1, 64, _W), lambda b, i: (b, 0, i, 0, 0)),
        ],
        out_specs=pl.BlockSpec(memory_space=pltpu.SMEM),
        out_shape=jax.ShapeDtypeStruct((1, 1), jnp.float32),
        scratch_shapes=[
            pltpu.VMEM((1, _W), jnp.float32),
            pltpu.VMEM((_B, 2, _K), jnp.float32),
        ],
    )(S4, S4, wx)

    return loss[0, 0]
